# Initial kernel scaffold; baseline (speedup 1.0000x reference)
import jax
import jax.numpy as jnp
from jax.experimental import pallas as pl


def kernel(indices, features, num_nodes, W1_w, W1_b, W2_w, W2_b, Watt_w, Watt_b, a):
    raise NotImplementedError("write your pallas kernel here")


# speedup vs baseline: 7.7488x; 7.7488x over previous
import functools

import jax
import jax.numpy as jnp
from jax import lax
from jax.experimental import pallas as pl
from jax.experimental.pallas import tpu as pltpu
from jax.experimental.pallas import tpu_sc as plsc

D = 128
BS = 512
K = 128
NC, NS = 2, 16
NW = NC * NS


def _pre_body(feat_ref, watt_ref, wattb_ref, a1_ref, a2_ref, w1_ref, w1b_ref,
              hmsg_ref, s_ref):
    f = feat_ref[...]
    ht = lax.dot_general(f, watt_ref[...], (((1,), (1,)), ((), ())),
                         preferred_element_type=jnp.float32) + wattb_ref[...]
    s1 = lax.dot_general(a1_ref[...], ht, (((1,), (1,)), ((), ())),
                         preferred_element_type=jnp.float32)
    s2 = lax.dot_general(a2_ref[...], ht, (((1,), (1,)), ((), ())),
                         preferred_element_type=jnp.float32)
    s_ref[0:1, :] = s1
    s_ref[1:2, :] = s2
    s_ref[2:8, :] = jnp.zeros((6, s1.shape[1]), jnp.float32)
    hmsg_ref[...] = lax.dot_general(f, w1_ref[...], (((1,), (1,)), ((), ())),
                                    preferred_element_type=jnp.float32) + w1b_ref[...]


def _tc_pre(featp, Watt_w, Watt_b, a1, a2, W1_w, W1_b):
    NP = featp.shape[0]
    grid = (NP // BS,)
    return pl.pallas_call(
        _pre_body,
        grid=grid,
        in_specs=[
            pl.BlockSpec((BS, D), lambda i: (i, 0)),
            pl.BlockSpec((D, D), lambda i: (0, 0)),
            pl.BlockSpec((1, D), lambda i: (0, 0)),
            pl.BlockSpec((1, D), lambda i: (0, 0)),
            pl.BlockSpec((1, D), lambda i: (0, 0)),
            pl.BlockSpec((D, D), lambda i: (0, 0)),
            pl.BlockSpec((1, D), lambda i: (0, 0)),
        ],
        out_specs=[
            pl.BlockSpec((BS, D), lambda i: (i, 0)),
            pl.BlockSpec((8, BS), lambda i: (0, i)),
        ],
        out_shape=[
            jax.ShapeDtypeStruct((NP, D), jnp.float32),
            jax.ShapeDtypeStruct((8, NP), jnp.float32),
        ],
    )(featp, Watt_w, Watt_b, a1, a2, W1_w, W1_b)


def _sc_edge_call(src, dst, s_out, hmsg, NP, E, EPW):
    cpt = EPW // K
    rows_per_tile = NP // NS
    mesh = plsc.VectorSubcoreMesh(core_axis_name="c", subcore_axis_name="s")

    def body(src_hbm, dst_hbm, s_hbm, hmsg_hbm, acc_out, den_out,
             s1_v, s2_v, den_v, srcb, dstb, w_v, rows_v, acc_sh, sem):
        c = lax.axis_index("c")
        s = lax.axis_index("s")
        wid = s * NC + c
        ebase = wid * EPW

        pltpu.sync_copy(s_hbm.at[0], s1_v)
        pltpu.sync_copy(s_hbm.at[1], s2_v)

        def _zden(i, _):
            den_v[pl.ds(i * 16, 16)] = jnp.zeros((16,), jnp.float32)
            return _
        lax.fori_loop(0, NP // 16, _zden, 0)

        def _zrow(i, _):
            for j in range(D // 16):
                rows_v[i, pl.ds(j * 16, 16)] = jnp.zeros((16,), jnp.float32)
            return _
        lax.fori_loop(0, K, _zrow, 0)
        for t in range(rows_per_tile // K):
            pltpu.sync_copy(rows_v, acc_sh.at[pl.ds(s * rows_per_tile + t * K, K)])
        plsc.subcore_barrier()

        iota16 = lax.broadcasted_iota(jnp.int32, (16,), 0)

        def chunk_body(ci, _):
            base = ebase + ci * K
            pltpu.sync_copy(src_hbm.at[pl.ds(base, K)], srcb)
            pltpu.sync_copy(dst_hbm.at[pl.ds(base, K)], dstb)
            cp = pltpu.async_copy(hmsg_hbm.at[srcb], rows_v, sem)
            for j in range(K // 16):
                si = srcb[pl.ds(j * 16, 16)]
                di = dstb[pl.ds(j * 16, 16)]
                z = plsc.load_gather(s1_v, [si]) + plsc.load_gather(s2_v, [di])
                z = jnp.where(z >= 0.0, z, 0.2 * z)
                w = jnp.exp(z)
                gid = base + j * 16 + iota16
                w = jnp.where(gid < E, w, 0.0)
                w_v[pl.ds(j * 16, 16)] = w
                plsc.addupdate_scatter(den_v, [di], w)
            cp.wait()

            def scale(k, _s):
                wk = plsc.load_gather(w_v, [lax.broadcast(k, (16,))])
                for j in range(D // 16):
                    rows_v[k, pl.ds(j * 16, 16)] = rows_v[k, pl.ds(j * 16, 16)] * wk
                return _s
            lax.fori_loop(0, K, scale, 0)
            pltpu.sync_copy(rows_v, acc_sh.at[dstb], add=True)
            return _
        lax.fori_loop(0, cpt, chunk_body, 0)

        plsc.subcore_barrier()
        pltpu.sync_copy(acc_sh.at[pl.ds(s * rows_per_tile, rows_per_tile)],
                        acc_out.at[c, pl.ds(s * rows_per_tile, rows_per_tile)])
        pltpu.sync_copy(den_v, den_out.at[wid])

    fn = pl.kernel(
        body,
        out_type=[
            jax.ShapeDtypeStruct((NC, NP, D), jnp.float32),
            jax.ShapeDtypeStruct((NW, NP), jnp.float32),
        ],
        mesh=mesh,
        compiler_params=pltpu.CompilerParams(needs_layout_passes=False),
        scratch_types=[
            pltpu.VMEM((NP,), jnp.float32),
            pltpu.VMEM((NP,), jnp.float32),
            pltpu.VMEM((NP,), jnp.float32),
            pltpu.VMEM((K,), jnp.int32),
            pltpu.VMEM((K,), jnp.int32),
            pltpu.VMEM((K,), jnp.float32),
            pltpu.VMEM((K, D), jnp.float32),
            pltpu.VMEM_SHARED((NP, D), jnp.float32),
            pltpu.SemaphoreType.DMA,
        ],
    )
    return fn(src, dst, s_out, hmsg)


def _post_body(acc_ref, den_ref, feat_ref, w2_ref, w2b_ref, out_ref):
    acc = acc_ref[0] + acc_ref[1]
    den = jnp.sum(den_ref[...], axis=0)[:, None]
    h = acc / (den + 1e-9)
    f = feat_ref[...]
    w2p = lax.dot_general(f * h, w2_ref[...], (((1,), (1,)), ((), ())),
                          preferred_element_type=jnp.float32) + w2b_ref[...]
    o = f + h + w2p
    out_ref[...] = jnp.where(o >= 0.0, o, 0.2 * o)


def _tc_post(acc, den, featp, W2_w, W2_b):
    NP = featp.shape[0]
    grid = (NP // BS,)
    return pl.pallas_call(
        _post_body,
        grid=grid,
        in_specs=[
            pl.BlockSpec((NC, BS, D), lambda i: (0, i, 0)),
            pl.BlockSpec((NW, BS), lambda i: (0, i)),
            pl.BlockSpec((BS, D), lambda i: (i, 0)),
            pl.BlockSpec((D, D), lambda i: (0, 0)),
            pl.BlockSpec((1, D), lambda i: (0, 0)),
        ],
        out_specs=pl.BlockSpec((BS, D), lambda i: (i, 0)),
        out_shape=jax.ShapeDtypeStruct((NP, D), jnp.float32),
    )(acc, den, featp, W2_w, W2_b)


def kernel(indices, features, num_nodes, W1_w, W1_b, W2_w, W2_b, Watt_w, Watt_b, a):
    N = features.shape[0]
    E = indices.shape[1]
    NP = -(-N // BS) * BS
    cpt = -(-E // (NW * K))
    EPW = cpt * K
    EP = EPW * NW

    src = jnp.pad(indices[0].astype(jnp.int32), (0, EP - E))
    dst = jnp.pad(indices[1].astype(jnp.int32), (0, EP - E))
    featp = jnp.pad(features.astype(jnp.float32), ((0, NP - N), (0, 0)))
    a1 = a[:D, 0].reshape(1, D).astype(jnp.float32)
    a2 = a[D:, 0].reshape(1, D).astype(jnp.float32)

    hmsg, s_out = _tc_pre(featp, Watt_w, Watt_b.reshape(1, D), a1, a2,
                          W1_w, W1_b.reshape(1, D))
    acc, den = _sc_edge_call(src, dst, s_out, hmsg, NP, E, EPW)
    out = _tc_post(acc, den, featp, W2_w, W2_b.reshape(1, D))
    return out[:N]


# speedup vs baseline: 10.5069x; 1.3559x over previous
import functools

import jax
import jax.numpy as jnp
from jax import lax
from jax.experimental import pallas as pl
from jax.experimental.pallas import tpu as pltpu
from jax.experimental.pallas import tpu_sc as plsc

D = 128
BS = 512
K = 64
NC, NS = 2, 16
NW = NC * NS


def _pre_body(feat_ref, watt_ref, wattb_ref, a1_ref, a2_ref, w1_ref, w1b_ref,
              hmsg_ref, s_ref):
    f = feat_ref[...]
    ht = lax.dot_general(f, watt_ref[...], (((1,), (1,)), ((), ())),
                         preferred_element_type=jnp.float32) + wattb_ref[...]
    s1 = lax.dot_general(a1_ref[...], ht, (((1,), (1,)), ((), ())),
                         preferred_element_type=jnp.float32)
    s2 = lax.dot_general(a2_ref[...], ht, (((1,), (1,)), ((), ())),
                         preferred_element_type=jnp.float32)
    s_ref[0:1, :] = s1
    s_ref[1:2, :] = s2
    s_ref[2:8, :] = jnp.zeros((6, s1.shape[1]), jnp.float32)
    hmsg_ref[...] = lax.dot_general(f, w1_ref[...], (((1,), (1,)), ((), ())),
                                    preferred_element_type=jnp.float32) + w1b_ref[...]


def _tc_pre(featp, Watt_w, Watt_b, a1, a2, W1_w, W1_b):
    NP = featp.shape[0]
    grid = (NP // BS,)
    return pl.pallas_call(
        _pre_body,
        grid=grid,
        in_specs=[
            pl.BlockSpec((BS, D), lambda i: (i, 0)),
            pl.BlockSpec((D, D), lambda i: (0, 0)),
            pl.BlockSpec((1, D), lambda i: (0, 0)),
            pl.BlockSpec((1, D), lambda i: (0, 0)),
            pl.BlockSpec((1, D), lambda i: (0, 0)),
            pl.BlockSpec((D, D), lambda i: (0, 0)),
            pl.BlockSpec((1, D), lambda i: (0, 0)),
        ],
        out_specs=[
            pl.BlockSpec((BS, D), lambda i: (i, 0)),
            pl.BlockSpec((8, BS), lambda i: (0, i)),
        ],
        out_shape=[
            jax.ShapeDtypeStruct((NP, D), jnp.float32),
            jax.ShapeDtypeStruct((8, NP), jnp.float32),
        ],
    )(featp, Watt_w, Watt_b, a1, a2, W1_w, W1_b)


def _sc_edge_call(src, dst, s_out, hmsg, NP, E, EPW):
    cpt = EPW // K
    npairs = cpt // 2
    rows_per_tile = NP // NS
    mesh = plsc.VectorSubcoreMesh(core_axis_name="c", subcore_axis_name="s")

    def body(src_hbm, dst_hbm, s_hbm, hmsg_hbm, acc_out, den_out,
             s1_v, s2_v, den_v, sb0, db0, sb1, db1, w0_v, w1_v, rows0, rows1,
             acc_sh, sem0, sem1):
        c = lax.axis_index("c")
        s = lax.axis_index("s")
        wid = s * NC + c
        ebase = wid * EPW

        pltpu.sync_copy(s_hbm.at[0], s1_v)
        pltpu.sync_copy(s_hbm.at[1], s2_v)

        def _zden(i, _):
            den_v[pl.ds(i * 16, 16)] = jnp.zeros((16,), jnp.float32)
            return _
        lax.fori_loop(0, NP // 16, _zden, 0)

        def _zrow(i, _):
            for j in range(D // 16):
                rows0[i, pl.ds(j * 16, 16)] = jnp.zeros((16,), jnp.float32)
            return _
        lax.fori_loop(0, K, _zrow, 0)
        for t in range(rows_per_tile // K):
            pltpu.sync_copy(rows0, acc_sh.at[pl.ds(s * rows_per_tile + t * K, K)])
        plsc.subcore_barrier()

        iota16 = lax.broadcasted_iota(jnp.int32, (16,), 0)

        def scalar_phase(base, sb, db, w_v):
            for j in range(K // 16):
                si = sb[pl.ds(j * 16, 16)]
                di = db[pl.ds(j * 16, 16)]
                z = plsc.load_gather(s1_v, [si]) + plsc.load_gather(s2_v, [di])
                z = jnp.where(z >= 0.0, z, 0.2 * z)
                w = jnp.exp(z)
                gid = base + j * 16 + iota16
                w = jnp.where(gid < E, w, 0.0)
                w_v[pl.ds(j * 16, 16)] = w
                plsc.addupdate_scatter(den_v, [di], w)

        def scale_rows(rows, w_v):
            def scale(k, _s):
                for u in range(4):
                    kk = k * 4 + u
                    wk = plsc.load_gather(w_v, [lax.broadcast(kk, (16,))])
                    for j in range(D // 16):
                        rows[kk, pl.ds(j * 16, 16)] = rows[kk, pl.ds(j * 16, 16)] * wk
                return _s
            lax.fori_loop(0, K // 4, scale, 0)

        pltpu.sync_copy(src_hbm.at[pl.ds(ebase, K)], sb0)
        pltpu.sync_copy(dst_hbm.at[pl.ds(ebase, K)], db0)
        pltpu.async_copy(hmsg_hbm.at[sb0], rows0, sem0)

        def pair_body(i, _):
            b0 = ebase + (2 * i) * K
            b1 = b0 + K
            pltpu.sync_copy(src_hbm.at[pl.ds(b1, K)], sb1)
            pltpu.sync_copy(dst_hbm.at[pl.ds(b1, K)], db1)
            pltpu.async_copy(hmsg_hbm.at[sb1], rows1, sem1)
            scalar_phase(b0, sb0, db0, w0_v)
            pltpu.make_async_copy(hmsg_hbm.at[sb0], rows0, sem0).wait()
            scale_rows(rows0, w0_v)
            pltpu.sync_copy(rows0, acc_sh.at[db0], add=True)

            @pl.when(i + 1 < npairs)
            def _fire_next():
                b2 = b1 + K
                pltpu.sync_copy(src_hbm.at[pl.ds(b2, K)], sb0)
                pltpu.sync_copy(dst_hbm.at[pl.ds(b2, K)], db0)
                pltpu.async_copy(hmsg_hbm.at[sb0], rows0, sem0)

            scalar_phase(b1, sb1, db1, w1_v)
            pltpu.make_async_copy(hmsg_hbm.at[sb1], rows1, sem1).wait()
            scale_rows(rows1, w1_v)
            pltpu.sync_copy(rows1, acc_sh.at[db1], add=True)
            return _
        lax.fori_loop(0, npairs, pair_body, 0)

        plsc.subcore_barrier()
        pltpu.sync_copy(acc_sh.at[pl.ds(s * rows_per_tile, rows_per_tile)],
                        acc_out.at[c, pl.ds(s * rows_per_tile, rows_per_tile)])
        pltpu.sync_copy(den_v, den_out.at[wid])

    fn = pl.kernel(
        body,
        out_type=[
            jax.ShapeDtypeStruct((NC, NP, D), jnp.float32),
            jax.ShapeDtypeStruct((NW, NP), jnp.float32),
        ],
        mesh=mesh,
        compiler_params=pltpu.CompilerParams(needs_layout_passes=False),
        scratch_types=[
            pltpu.VMEM((NP,), jnp.float32),
            pltpu.VMEM((NP,), jnp.float32),
            pltpu.VMEM((NP,), jnp.float32),
            pltpu.VMEM((K,), jnp.int32),
            pltpu.VMEM((K,), jnp.int32),
            pltpu.VMEM((K,), jnp.int32),
            pltpu.VMEM((K,), jnp.int32),
            pltpu.VMEM((K,), jnp.float32),
            pltpu.VMEM((K,), jnp.float32),
            pltpu.VMEM((K, D), jnp.float32),
            pltpu.VMEM((K, D), jnp.float32),
            pltpu.VMEM_SHARED((NP, D), jnp.float32),
            pltpu.SemaphoreType.DMA,
            pltpu.SemaphoreType.DMA,
        ],
    )
    return fn(src, dst, s_out, hmsg)


def _post_body(acc_ref, den_ref, feat_ref, w2_ref, w2b_ref, out_ref):
    acc = acc_ref[0] + acc_ref[1]
    den = jnp.sum(den_ref[...], axis=0)[:, None]
    h = acc / (den + 1e-9)
    f = feat_ref[...]
    w2p = lax.dot_general(f * h, w2_ref[...], (((1,), (1,)), ((), ())),
                          preferred_element_type=jnp.float32) + w2b_ref[...]
    o = f + h + w2p
    out_ref[...] = jnp.where(o >= 0.0, o, 0.2 * o)


def _tc_post(acc, den, featp, W2_w, W2_b):
    NP = featp.shape[0]
    grid = (NP // BS,)
    return pl.pallas_call(
        _post_body,
        grid=grid,
        in_specs=[
            pl.BlockSpec((NC, BS, D), lambda i: (0, i, 0)),
            pl.BlockSpec((NW, BS), lambda i: (0, i)),
            pl.BlockSpec((BS, D), lambda i: (i, 0)),
            pl.BlockSpec((D, D), lambda i: (0, 0)),
            pl.BlockSpec((1, D), lambda i: (0, 0)),
        ],
        out_specs=pl.BlockSpec((BS, D), lambda i: (i, 0)),
        out_shape=jax.ShapeDtypeStruct((NP, D), jnp.float32),
    )(acc, den, featp, W2_w, W2_b)


def kernel(indices, features, num_nodes, W1_w, W1_b, W2_w, W2_b, Watt_w, Watt_b, a):
    N = features.shape[0]
    E = indices.shape[1]
    NP = -(-N // BS) * BS
    cpt = -(-E // (NW * K))
    cpt = cpt + (cpt % 2)
    EPW = cpt * K
    EP = EPW * NW

    idxp = jnp.pad(indices.astype(jnp.int32), ((0, 0), (0, EP - E)))
    featp = jnp.pad(features.astype(jnp.float32), ((0, NP - N), (0, 0)))
    a1 = a[:D, 0].reshape(1, D).astype(jnp.float32)
    a2 = a[D:, 0].reshape(1, D).astype(jnp.float32)

    hmsg, s_out = _tc_pre(featp, Watt_w, Watt_b.reshape(1, D), a1, a2,
                          W1_w, W1_b.reshape(1, D))
    acc, den = _sc_edge_call(idxp[0], idxp[1], s_out, hmsg, NP, E, EPW)
    out = _tc_post(acc, den, featp, W2_w, W2_b.reshape(1, D))
    return out[:N]
